# baseline (speedup 1.0000x reference)
import numpy as np

import jax
import jax.numpy as jnp
from jax.experimental import pallas as pl
from jax.experimental.pallas import tpu as pltpu

_N = 100000
_BATCH = 128
_MID = 4
_B = 2
_R = _B * _MID
_W = 2048
_NFULL = _N // _W
_REM = _N - _NFULL * _W

_TINY = np.float32(np.finfo(np.float32).tiny)


def _threefry_bits(x1):
    k0, k1 = 0, 42
    k2 = k0 ^ k1 ^ 0x1BD11BDA
    ks = (k0, k1, k2)
    rot = ((13, 15, 26, 6), (17, 29, 16, 24))
    x0 = x1
    x1 = ((x1 << 13) | (x1 >> 19)) ^ x0
    first = True
    for grp in range(5):
        for r in rot[grp % 2]:
            if first:
                first = False
                continue
            x0 = x0 + x1
            x1 = (x1 << r) | (x1 >> (32 - r))
            x1 = x1 ^ x0
        c0 = ks[(grp + 1) % 3] & 0xFFFFFFFF
        c1 = (ks[(grp + 2) % 3] + grp + 1) & 0xFFFFFFFF
        if c0:
            x0 = x0 + jnp.uint32(c0)
        x1 = x1 + jnp.uint32(c1)
    return x0 ^ x1


def _chunk_score(p_chunk, ctr):
    bits = _threefry_bits(ctr)
    fb = (bits >> 9) | jnp.uint32(0x3F800000)
    f = jax.lax.bitcast_convert_type(fb, jnp.float32) - jnp.float32(1.0)
    u = jnp.maximum(_TINY, f + _TINY)
    g = -jnp.log(-jnp.log(u))
    return g + jnp.log(p_chunk)


def _sample_kernel(p_ref, out_ref):
    i = pl.program_id(0)
    row0 = (i * _R).astype(jnp.uint32)
    rowbase = (jax.lax.broadcasted_iota(jnp.uint32, (_R, 1), 0)
               + row0) * jnp.uint32(_N)
    lane = jax.lax.broadcasted_iota(jnp.uint32, (_R, _W), 1)
    ctr0 = rowbase + lane + jnp.uint32(42)

    m_lane = jnp.full((_R, _W), -jnp.inf, jnp.float32)
    j_lane = jnp.zeros((_R, _W), jnp.int32)
    for j in range(_NFULL):
        p_chunk = jnp.reshape(p_ref[:, :, j * _W:(j + 1) * _W], (_R, _W))
        score = _chunk_score(p_chunk, ctr0 + jnp.uint32(j * _W))
        better = score > m_lane
        m_lane = jnp.where(better, score, m_lane)
        j_lane = jnp.where(better, jnp.int32(j), j_lane)

    m = jnp.max(m_lane, axis=1, keepdims=True)
    flat = j_lane * _W + jax.lax.broadcasted_iota(jnp.int32, (_R, _W), 1)
    bi = jnp.min(jnp.where(m_lane == m, flat, jnp.int32(_N)), axis=1,
                 keepdims=True)

    c0 = _NFULL * _W
    lane_r = jax.lax.broadcasted_iota(jnp.uint32, (_R, _REM), 1)
    p_rem = jnp.reshape(p_ref[:, :, c0:], (_R, _REM))
    score = _chunk_score(p_rem, rowbase + lane_r + jnp.uint32(c0 + 42))
    mr = jnp.max(score, axis=1, keepdims=True)
    bir = jnp.min(
        jnp.where(score == mr, lane_r.astype(jnp.int32) + c0, jnp.int32(_N)),
        axis=1, keepdims=True)
    better = mr > m
    bi = jnp.where(better, bir, bi)

    icol = jax.lax.broadcasted_iota(jnp.int32, (_R, _W), 1)
    for j in range(_NFULL):
        oh = (icol == bi - j * _W).astype(jnp.float32)
        out_ref[:, :, j * _W:(j + 1) * _W] = jnp.reshape(oh, (_B, _MID, _W))
    icol_r = jax.lax.broadcasted_iota(jnp.int32, (_R, _REM), 1)
    oh_r = (icol_r == bi - c0).astype(jnp.float32)
    out_ref[:, :, c0:] = jnp.reshape(oh_r, (_B, _MID, _REM))


def kernel(input):
    return pl.pallas_call(
        _sample_kernel,
        grid=(_BATCH // _B,),
        in_specs=[pl.BlockSpec((_B, _MID, _N), lambda i: (i, 0, 0))],
        out_specs=pl.BlockSpec((_B, _MID, _N), lambda i: (i, 0, 0)),
        out_shape=jax.ShapeDtypeStruct((_BATCH, _MID, _N), jnp.float32),
        compiler_params=pltpu.CompilerParams(
            dimension_semantics=("parallel",),
        ),
    )(input)


# speedup vs baseline: 1.0385x; 1.0385x over previous
import numpy as np

import jax
import jax.numpy as jnp
from jax.experimental import pallas as pl
from jax.experimental.pallas import tpu as pltpu

_N = 100000
_BATCH = 128
_MID = 4
_B = 2
_R = _B * _MID
_W = 512
_NFULL = _N // _W
_REM = _N - _NFULL * _W

_TINY = np.float32(np.finfo(np.float32).tiny)


def _threefry_bits(x1):
    k0, k1 = 0, 42
    k2 = k0 ^ k1 ^ 0x1BD11BDA
    ks = (k0, k1, k2)
    rot = ((13, 15, 26, 6), (17, 29, 16, 24))
    x0 = x1
    x1 = ((x1 << 13) | (x1 >> 19)) ^ x0
    first = True
    for grp in range(5):
        for r in rot[grp % 2]:
            if first:
                first = False
                continue
            x0 = x0 + x1
            x1 = (x1 << r) | (x1 >> (32 - r))
            x1 = x1 ^ x0
        c0 = ks[(grp + 1) % 3] & 0xFFFFFFFF
        c1 = (ks[(grp + 2) % 3] + grp + 1) & 0xFFFFFFFF
        if c0:
            x0 = x0 + jnp.uint32(c0)
        x1 = x1 + jnp.uint32(c1)
    return x0 ^ x1


def _chunk_score(p_chunk, ctr):
    bits = _threefry_bits(ctr)
    fb = (bits >> 9) | jnp.uint32(0x3F800000)
    f = jax.lax.bitcast_convert_type(fb, jnp.float32) - jnp.float32(1.0)
    u = jnp.maximum(_TINY, f + _TINY)
    g = -jnp.log(-jnp.log(u))
    return g + jnp.log(p_chunk)


def _sample_kernel(p_ref, out_ref):
    i = pl.program_id(0)
    row0 = (i * _R).astype(jnp.uint32)
    rowbase = (jax.lax.broadcasted_iota(jnp.uint32, (_R, 1), 0)
               + row0) * jnp.uint32(_N)
    lane = jax.lax.broadcasted_iota(jnp.uint32, (_R, _W), 1)
    ctr0 = rowbase + lane + jnp.uint32(42)

    m_lane = jnp.full((_R, _W), -jnp.inf, jnp.float32)
    j_lane = jnp.zeros((_R, _W), jnp.int32)
    for j in range(_NFULL):
        p_chunk = jnp.reshape(p_ref[:, :, j * _W:(j + 1) * _W], (_R, _W))
        score = _chunk_score(p_chunk, ctr0 + jnp.uint32(j * _W))
        better = score > m_lane
        m_lane = jnp.where(better, score, m_lane)
        j_lane = jnp.where(better, jnp.int32(j), j_lane)

    m = jnp.max(m_lane, axis=1, keepdims=True)
    flat = j_lane * _W + jax.lax.broadcasted_iota(jnp.int32, (_R, _W), 1)
    bi = jnp.min(jnp.where(m_lane == m, flat, jnp.int32(_N)), axis=1,
                 keepdims=True)

    c0 = _NFULL * _W
    lane_r = jax.lax.broadcasted_iota(jnp.uint32, (_R, _REM), 1)
    p_rem = jnp.reshape(p_ref[:, :, c0:], (_R, _REM))
    score = _chunk_score(p_rem, rowbase + lane_r + jnp.uint32(c0 + 42))
    mr = jnp.max(score, axis=1, keepdims=True)
    bir = jnp.min(
        jnp.where(score == mr, lane_r.astype(jnp.int32) + c0, jnp.int32(_N)),
        axis=1, keepdims=True)
    better = mr > m
    bi = jnp.where(better, bir, bi)

    icol = jax.lax.broadcasted_iota(jnp.int32, (_R, _W), 1)
    for j in range(_NFULL):
        oh = (icol == bi - j * _W).astype(jnp.float32)
        out_ref[:, :, j * _W:(j + 1) * _W] = jnp.reshape(oh, (_B, _MID, _W))
    icol_r = jax.lax.broadcasted_iota(jnp.int32, (_R, _REM), 1)
    oh_r = (icol_r == bi - c0).astype(jnp.float32)
    out_ref[:, :, c0:] = jnp.reshape(oh_r, (_B, _MID, _REM))


def kernel(input):
    return pl.pallas_call(
        _sample_kernel,
        grid=(_BATCH // _B,),
        in_specs=[pl.BlockSpec((_B, _MID, _N), lambda i: (i, 0, 0))],
        out_specs=pl.BlockSpec((_B, _MID, _N), lambda i: (i, 0, 0)),
        out_shape=jax.ShapeDtypeStruct((_BATCH, _MID, _N), jnp.float32),
        compiler_params=pltpu.CompilerParams(
            dimension_semantics=("parallel",),
        ),
    )(input)
